# baseline (speedup 1.0000x reference)
import functools

import jax
import jax.numpy as jnp
from jax import lax
from jax.experimental import pallas as pl
from jax.experimental.pallas import tpu as pltpu
from jax.experimental.pallas import tpu_sc as plsc

N_NODES = 10000
HID = 128
NP = 10240
NC = 2
NS = 16
NW = NC * NS
SW = NP // NS
CH = 80
NBUF = 4
BLK = 1024


def _pre_body(x_ref, wk_ref, bk_ref, wv_ref, bv_ref, av_ref, xv_ref, al_ref):
    xb = x_ref[...]
    xv_ref[...] = (
        jnp.dot(xb, wv_ref[...], preferred_element_type=jnp.float32)
        + bv_ref[...]
    )
    wa = jnp.dot(wk_ref[...], av_ref[...], preferred_element_type=jnp.float32)
    ba = jnp.dot(bk_ref[...], av_ref[...], preferred_element_type=jnp.float32)
    al_ref[...] = (
        jnp.dot(xb, wa, preferred_element_type=jnp.float32) + ba
    )


def _tc_pre(hp, p):
    ind = hp.shape[1]
    av = p['att_r'].reshape(HID, 1)
    bk = p['bK'].reshape(1, HID)
    bv = p['bV'].reshape(1, HID)
    xv, al = pl.pallas_call(
        _pre_body,
        grid=(NP // BLK,),
        in_specs=[
            pl.BlockSpec((BLK, ind), lambda i: (i, 0)),
            pl.BlockSpec((ind, HID), lambda i: (0, 0)),
            pl.BlockSpec((1, HID), lambda i: (0, 0)),
            pl.BlockSpec((ind, HID), lambda i: (0, 0)),
            pl.BlockSpec((1, HID), lambda i: (0, 0)),
            pl.BlockSpec((HID, 1), lambda i: (0, 0)),
        ],
        out_specs=[
            pl.BlockSpec((BLK, HID), lambda i: (i, 0)),
            pl.BlockSpec((BLK, 1), lambda i: (i, 0)),
        ],
        out_shape=[
            jax.ShapeDtypeStruct((NP, HID), jnp.float32),
            jax.ShapeDtypeStruct((NP, 1), jnp.float32),
        ],
    )(hp, p['WK'], bk, p['WV'], bv, av)
    return xv, al.reshape(NP)


def _ln_rows(o, g, b):
    m = jnp.mean(o, axis=-1, keepdims=True)
    v = jnp.mean((o - m) ** 2, axis=-1, keepdims=True)
    return (o - m) / jnp.sqrt(v + 1e-5) * g + b


def _post_body(p0_ref, p1_ref, ar_ref, g0_ref, b0_ref, w1_ref, c1_ref,
               w2_ref, c2_ref, g1_ref, b1_ref, out_ref):
    o = p0_ref[...] + p1_ref[...] + ar_ref[...]
    o = _ln_rows(o, g0_ref[...], b0_ref[...])
    ff = jnp.dot(
        jnp.maximum(
            jnp.dot(o, w1_ref[...], preferred_element_type=jnp.float32)
            + c1_ref[...], 0.0),
        w2_ref[...], preferred_element_type=jnp.float32) + c2_ref[...]
    o2 = o + jnp.maximum(ff, 0.0)
    out_ref[...] = jnp.maximum(_ln_rows(o2, g1_ref[...], b1_ref[...]), 0.0)


def _tc_post(p0, p1, p):
    row = lambda a: a.reshape(1, HID)
    full = lambda i: (0, 0)
    blk = lambda i: (i, 0)
    return pl.pallas_call(
        _post_body,
        grid=(NP // BLK,),
        in_specs=[
            pl.BlockSpec((BLK, HID), blk),
            pl.BlockSpec((BLK, HID), blk),
            pl.BlockSpec((1, HID), full),
            pl.BlockSpec((1, HID), full),
            pl.BlockSpec((1, HID), full),
            pl.BlockSpec((HID, HID), full),
            pl.BlockSpec((1, HID), full),
            pl.BlockSpec((HID, HID), full),
            pl.BlockSpec((1, HID), full),
            pl.BlockSpec((1, HID), full),
            pl.BlockSpec((1, HID), full),
        ],
        out_specs=pl.BlockSpec((BLK, HID), blk),
        out_shape=jax.ShapeDtypeStruct((NP, HID), jnp.float32),
    )(p0, p1, row(p['att_r'].reshape(HID)), row(p['ln0_g']), row(p['ln0_b']),
      p['W1'], row(p['b1']), p['W2'], row(p['b2']),
      row(p['ln1_g']), row(p['ln1_b']))


def _sc_softmax(alpha, sidx, didx, EWP):
    E = sidx.shape[0]
    EW2 = E // NS
    EH = EW2 // NC
    PADN = EWP - EH
    mesh = plsc.VectorSubcoreMesh(core_axis_name="c", subcore_axis_name="s")

    @functools.partial(
        pl.kernel,
        out_type=jax.ShapeDtypeStruct((NW * EWP,), jnp.float32),
        mesh=mesh,
        compiler_params=pltpu.CompilerParams(needs_layout_passes=False),
        scratch_types=[
            pltpu.VMEM((NP,), jnp.float32),
            pltpu.VMEM((EW2,), jnp.int32),
            pltpu.VMEM((EW2,), jnp.int32),
            pltpu.VMEM((EW2,), jnp.float32),
            pltpu.VMEM((NP,), jnp.float32),
            pltpu.VMEM((NS, SW), jnp.float32),
            pltpu.VMEM((SW,), jnp.float32),
            pltpu.VMEM((PADN,), jnp.float32),
            pltpu.VMEM_SHARED((NS, NP), jnp.float32),
            pltpu.VMEM_SHARED((NP,), jnp.float32),
            pltpu.SemaphoreType.DMA,
        ],
    )
    def k(alpha_h, sidx_h, didx_h, attn_h,
          alpha_v, sidx_v, didx_v, aexp_v, dacc_v, redm_v, red_v, zpad_v,
          sp_all, sp_den, isem):
        cid = lax.axis_index("c")
        sid = lax.axis_index("s")
        base = sid * EW2
        lds = [pltpu.async_copy(alpha_h, alpha_v, isem),
               pltpu.async_copy(sidx_h.at[pl.ds(base, EW2)], sidx_v, isem),
               pltpu.async_copy(didx_h.at[pl.ds(base, EW2)], didx_v, isem)]

        def zbody(i, _):
            dacc_v[pl.ds(i * 16, 16)] = jnp.zeros((16,), jnp.float32)
            return 0
        lax.fori_loop(0, NP // 16, zbody, 0, unroll=8)
        for g in range(PADN // 16):
            zpad_v[pl.ds(g * 16, 16)] = jnp.zeros((16,), jnp.float32)
        for d in lds:
            d.wait()

        def ebody(g, _):
            s16 = sidx_v[pl.ds(g * 16, 16)]
            a16 = plsc.load_gather(alpha_v, [s16])
            a16 = jnp.where(a16 >= 0.0, a16, a16 * 0.2)
            x16 = jnp.exp(a16)
            aexp_v[pl.ds(g * 16, 16)] = x16
            d16 = didx_v[pl.ds(g * 16, 16)]
            plsc.addupdate_scatter(dacc_v, [d16], x16)
            return 0
        lax.fori_loop(0, EW2 // 16, ebody, 0, unroll=4)

        pltpu.sync_copy(dacc_v, sp_all.at[sid])
        plsc.subcore_barrier()

        rds = [pltpu.async_copy(
            sp_all.at[t, pl.ds(sid * SW, SW)], redm_v.at[t], isem)
            for t in range(NS)]
        for d in rds:
            d.wait()

        def abody(i, _):
            s = redm_v[0, pl.ds(i * 16, 16)]
            for t in range(1, NS):
                s = s + redm_v[t, pl.ds(i * 16, 16)]
            red_v[pl.ds(i * 16, 16)] = s
            return 0
        lax.fori_loop(0, SW // 16, abody, 0, unroll=4)
        pltpu.sync_copy(red_v, sp_den.at[pl.ds(sid * SW, SW)])
        plsc.subcore_barrier()
        pltpu.sync_copy(sp_den, dacc_v)

        off = cid * EH

        def fbody(g, _):
            x16 = aexp_v[pl.ds(off + g * 16, 16)]
            d16 = didx_v[pl.ds(off + g * 16, 16)]
            dn16 = plsc.load_gather(dacc_v, [d16])
            aexp_v[pl.ds(off + g * 16, 16)] = x16 / (dn16 + 1e-16)
            return 0
        lax.fori_loop(0, EH // 16, fbody, 0, unroll=4)
        blk = (2 * sid + cid) * EWP
        pltpu.sync_copy(
            aexp_v.at[pl.ds(off, EH)], attn_h.at[pl.ds(blk, EH)])
        pltpu.sync_copy(zpad_v, attn_h.at[pl.ds(blk + EH, PADN)])

    return k(alpha, sidx, didx)


def _sc_scatter(xv, attn, sidx, didx):
    EWP = sidx.shape[0] // NW
    NCHK = EWP // CH
    RPB = 4 * NBUF
    CPB = RPB * CH
    mesh = plsc.VectorSubcoreMesh(core_axis_name="c", subcore_axis_name="s")

    @functools.partial(
        pl.kernel,
        out_type=jax.ShapeDtypeStruct((NC * NP, HID), jnp.float32),
        mesh=mesh,
        compiler_params=pltpu.CompilerParams(needs_layout_passes=False),
        scratch_types=(
            [pltpu.VMEM((CH, HID), jnp.float32) for _ in range(NBUF)]
            + [pltpu.VMEM((CPB,), jnp.int32),
               pltpu.VMEM((CPB,), jnp.int32),
               pltpu.VMEM((CPB,), jnp.float32)]
            + [pltpu.VMEM((CH,), jnp.int32) for _ in range(2 * NBUF)]
            + [pltpu.VMEM_SHARED((NP, HID), jnp.float32)]
            + [pltpu.SemaphoreType.DMA for _ in range(2 * NBUF + 1)]),
    )
    def k(xv_h, attn_h, sidx_h, didx_h, outp_h, *rest):
        rows = rest[:NBUF]
        ibuf = rest[NBUF]
        dbuf = rest[NBUF + 1]
        abuf = rest[NBUF + 2]
        didxw = rest[NBUF + 3:2 * NBUF + 3]
        cidxw = rest[2 * NBUF + 3:3 * NBUF + 3]
        sp_out = rest[3 * NBUF + 3]
        gsem = rest[3 * NBUF + 4:4 * NBUF + 4]
        ssem = rest[4 * NBUF + 4:5 * NBUF + 4]
        isem = rest[5 * NBUF + 4]
        cid = lax.axis_index("c")
        sid = lax.axis_index("s")
        base = (cid * NS + sid) * EWP

        def zrow(r, _):
            for f in range(HID // 16):
                rows[0][r, pl.ds(f * 16, 16)] = jnp.zeros((16,), jnp.float32)
            return 0
        lax.fori_loop(0, CH, zrow, 0, unroll=4)
        for kk in range(SW // CH):
            pltpu.sync_copy(rows[0], sp_out.at[pl.ds(sid * SW + kk * CH, CH)])
        plsc.subcore_barrier()

        def set_didxw(r, b):
            for g in range(CH // 16):
                didxw[b][pl.ds(g * 16, 16)] = (
                    dbuf[pl.ds(r * CH + g * 16, 16)])

        def set_cidxw(r, b):
            for g in range(CH // 16):
                cidxw[b][pl.ds(g * 16, 16)] = (
                    ibuf[pl.ds(r * CH + g * 16, 16)])

        def scale(r, b):
            def sbody(j, _):
                av = plsc.load_gather(
                    abuf, [jnp.zeros((16,), jnp.int32) + (r * CH + j)])
                for f in range(HID // 16):
                    rows[b][j, pl.ds(f * 16, 16)] = (
                        rows[b][j, pl.ds(f * 16, 16)] * av)
                return 0
            lax.fori_loop(0, CH, sbody, 0, unroll=8)

        def body(p, _):
            eb = base + p * CPB
            ids = [pltpu.async_copy(sidx_h.at[pl.ds(eb, CPB)], ibuf, isem),
                   pltpu.async_copy(didx_h.at[pl.ds(eb, CPB)], dbuf, isem),
                   pltpu.async_copy(attn_h.at[pl.ds(eb, CPB)], abuf, isem)]
            for d in ids:
                d.wait()
            sds = None
            for w in range(RPB // NBUF):
                gds = []
                for b in range(NBUF):
                    if sds is not None:
                        sds[b].wait()
                    set_cidxw(w * NBUF + b, b)
                    gds.append(pltpu.async_copy(
                        xv_h.at[cidxw[b]], rows[b], gsem[b]))
                sds = []
                for b in range(NBUF):
                    gds[b].wait()
                    scale(w * NBUF + b, b)
                    set_didxw(w * NBUF + b, b)
                    sds.append(pltpu.async_copy(
                        rows[b], sp_out.at[didxw[b]], ssem[b], add=True))
            for d in sds:
                d.wait()
            return 0
        lax.fori_loop(0, NCHK // RPB, body, 0)

        plsc.subcore_barrier()
        pltpu.sync_copy(
            sp_out.at[pl.ds(sid * SW, SW)],
            outp_h.at[pl.ds(cid * NP + sid * SW, SW)])

    return k(xv, attn, sidx, didx)


def kernel(x, edge_index, params):
    src = edge_index[0]
    he = edge_index[1] - jnp.min(edge_index[1])
    E = src.shape[0]
    EW = E // NW
    EWP = ((EW // CH + 7) // 8 * 8) * CH

    def _pad(a, padvals):
        a2 = a.reshape(NW, EW)
        pb = jnp.broadcast_to(padvals, (NW, EWP - EW)).astype(a.dtype)
        return jnp.concatenate([a2, pb], axis=1).reshape(NW * EWP)

    gspread = jnp.arange(EWP - EW, dtype=jnp.int32) % N_NODES
    sspread = NP - (EWP - EW) + jnp.arange(EWP - EW, dtype=jnp.int32)
    src_p = _pad(src, gspread)
    he_p = _pad(he, gspread)
    src_d = _pad(src, sspread)
    he_d = _pad(he, sspread)
    hp = jnp.zeros((NP, x.shape[1]), x.dtype).at[:N_NODES].set(x)
    for li, p in enumerate(params):
        s_ids, d_ids = (src, he) if li % 2 == 0 else (he, src)
        s2, d2 = (src_p, he_d) if li % 2 == 0 else (he_p, src_d)
        xv, alpha = _tc_pre(hp, p)
        attn = _sc_softmax(alpha, s_ids, d_ids, EWP)
        outp = _sc_scatter(xv, attn, s2, d2)
        hp = _tc_post(outp[:NP], outp[NP:], p)
    return hp[:N_NODES]


# speedup vs baseline: 1.0440x; 1.0440x over previous
import functools

import jax
import jax.numpy as jnp
from jax import lax
from jax.experimental import pallas as pl
from jax.experimental.pallas import tpu as pltpu
from jax.experimental.pallas import tpu_sc as plsc

N_NODES = 10000
HID = 128
NP = 10240
NC = 2
NS = 16
NW = NC * NS
SW = NP // NS
CH = 80
NBUF = 4
BLK = 1024


def _pre_body(x_ref, wk_ref, bk_ref, wv_ref, bv_ref, av_ref, xv_ref, al_ref):
    xb = x_ref[...]
    xv_ref[...] = (
        jnp.dot(xb, wv_ref[...], preferred_element_type=jnp.float32)
        + bv_ref[...]
    )
    wa = jnp.dot(wk_ref[...], av_ref[...], preferred_element_type=jnp.float32)
    ba = jnp.dot(bk_ref[...], av_ref[...], preferred_element_type=jnp.float32)
    al_ref[...] = (
        jnp.dot(xb, wa, preferred_element_type=jnp.float32) + ba
    )


def _tc_pre(hp, p):
    ind = hp.shape[1]
    av = p['att_r'].reshape(HID, 1)
    bk = p['bK'].reshape(1, HID)
    bv = p['bV'].reshape(1, HID)
    xv, al = pl.pallas_call(
        _pre_body,
        grid=(NP // BLK,),
        in_specs=[
            pl.BlockSpec((BLK, ind), lambda i: (i, 0)),
            pl.BlockSpec((ind, HID), lambda i: (0, 0)),
            pl.BlockSpec((1, HID), lambda i: (0, 0)),
            pl.BlockSpec((ind, HID), lambda i: (0, 0)),
            pl.BlockSpec((1, HID), lambda i: (0, 0)),
            pl.BlockSpec((HID, 1), lambda i: (0, 0)),
        ],
        out_specs=[
            pl.BlockSpec((BLK, HID), lambda i: (i, 0)),
            pl.BlockSpec((BLK, 1), lambda i: (i, 0)),
        ],
        out_shape=[
            jax.ShapeDtypeStruct((NP, HID), jnp.float32),
            jax.ShapeDtypeStruct((NP, 1), jnp.float32),
        ],
    )(hp, p['WK'], bk, p['WV'], bv, av)
    return xv, al.reshape(NP)


def _ln_rows(o, g, b):
    m = jnp.mean(o, axis=-1, keepdims=True)
    v = jnp.mean((o - m) ** 2, axis=-1, keepdims=True)
    return (o - m) / jnp.sqrt(v + 1e-5) * g + b


def _post_body(p0_ref, p1_ref, ar_ref, g0_ref, b0_ref, w1_ref, c1_ref,
               w2_ref, c2_ref, g1_ref, b1_ref, out_ref):
    o = p0_ref[...] + p1_ref[...] + ar_ref[...]
    o = _ln_rows(o, g0_ref[...], b0_ref[...])
    ff = jnp.dot(
        jnp.maximum(
            jnp.dot(o, w1_ref[...], preferred_element_type=jnp.float32)
            + c1_ref[...], 0.0),
        w2_ref[...], preferred_element_type=jnp.float32) + c2_ref[...]
    o2 = o + jnp.maximum(ff, 0.0)
    out_ref[...] = jnp.maximum(_ln_rows(o2, g1_ref[...], b1_ref[...]), 0.0)


def _post_pre_body(p0_ref, p1_ref, ar_ref, g0_ref, b0_ref, w1_ref, c1_ref,
                   w2_ref, c2_ref, g1_ref, b1_ref,
                   wk_ref, bk_ref, wv_ref, bv_ref, av_ref,
                   xv_ref, al_ref):
    o = p0_ref[...] + p1_ref[...] + ar_ref[...]
    o = _ln_rows(o, g0_ref[...], b0_ref[...])
    ff = jnp.dot(
        jnp.maximum(
            jnp.dot(o, w1_ref[...], preferred_element_type=jnp.float32)
            + c1_ref[...], 0.0),
        w2_ref[...], preferred_element_type=jnp.float32) + c2_ref[...]
    o2 = o + jnp.maximum(ff, 0.0)
    h = jnp.maximum(_ln_rows(o2, g1_ref[...], b1_ref[...]), 0.0)
    xv_ref[...] = (
        jnp.dot(h, wv_ref[...], preferred_element_type=jnp.float32)
        + bv_ref[...]
    )
    wa = jnp.dot(wk_ref[...], av_ref[...], preferred_element_type=jnp.float32)
    ba = jnp.dot(bk_ref[...], av_ref[...], preferred_element_type=jnp.float32)
    al_ref[...] = (
        jnp.dot(h, wa, preferred_element_type=jnp.float32) + ba
    )


def _tc_post_pre(p0, p1, p, pn):
    row = lambda a: a.reshape(1, HID)
    full = lambda i: (0, 0)
    blk = lambda i: (i, 0)
    xv, al = pl.pallas_call(
        _post_pre_body,
        grid=(NP // BLK,),
        in_specs=[
            pl.BlockSpec((BLK, HID), blk),
            pl.BlockSpec((BLK, HID), blk),
            pl.BlockSpec((1, HID), full),
            pl.BlockSpec((1, HID), full),
            pl.BlockSpec((1, HID), full),
            pl.BlockSpec((HID, HID), full),
            pl.BlockSpec((1, HID), full),
            pl.BlockSpec((HID, HID), full),
            pl.BlockSpec((1, HID), full),
            pl.BlockSpec((1, HID), full),
            pl.BlockSpec((1, HID), full),
            pl.BlockSpec((HID, HID), full),
            pl.BlockSpec((1, HID), full),
            pl.BlockSpec((HID, HID), full),
            pl.BlockSpec((1, HID), full),
            pl.BlockSpec((HID, 1), full),
        ],
        out_specs=[
            pl.BlockSpec((BLK, HID), blk),
            pl.BlockSpec((BLK, 1), blk),
        ],
        out_shape=[
            jax.ShapeDtypeStruct((NP, HID), jnp.float32),
            jax.ShapeDtypeStruct((NP, 1), jnp.float32),
        ],
    )(p0, p1, row(p['att_r'].reshape(HID)), row(p['ln0_g']), row(p['ln0_b']),
      p['W1'], row(p['b1']), p['W2'], row(p['b2']),
      row(p['ln1_g']), row(p['ln1_b']),
      pn['WK'], row(pn['bK']), pn['WV'], row(pn['bV']),
      pn['att_r'].reshape(HID, 1))
    return xv, al.reshape(NP)


def _tc_post(p0, p1, p):
    row = lambda a: a.reshape(1, HID)
    full = lambda i: (0, 0)
    blk = lambda i: (i, 0)
    return pl.pallas_call(
        _post_body,
        grid=(NP // BLK,),
        in_specs=[
            pl.BlockSpec((BLK, HID), blk),
            pl.BlockSpec((BLK, HID), blk),
            pl.BlockSpec((1, HID), full),
            pl.BlockSpec((1, HID), full),
            pl.BlockSpec((1, HID), full),
            pl.BlockSpec((HID, HID), full),
            pl.BlockSpec((1, HID), full),
            pl.BlockSpec((HID, HID), full),
            pl.BlockSpec((1, HID), full),
            pl.BlockSpec((1, HID), full),
            pl.BlockSpec((1, HID), full),
        ],
        out_specs=pl.BlockSpec((BLK, HID), blk),
        out_shape=jax.ShapeDtypeStruct((NP, HID), jnp.float32),
    )(p0, p1, row(p['att_r'].reshape(HID)), row(p['ln0_g']), row(p['ln0_b']),
      p['W1'], row(p['b1']), p['W2'], row(p['b2']),
      row(p['ln1_g']), row(p['ln1_b']))


def _sc_softmax(alpha, sidx, didx, EWP):
    E = sidx.shape[0]
    EW2 = E // NS
    EH = EW2 // NC
    PADN = EWP - EH
    mesh = plsc.VectorSubcoreMesh(core_axis_name="c", subcore_axis_name="s")

    @functools.partial(
        pl.kernel,
        out_type=jax.ShapeDtypeStruct((NW * EWP,), jnp.float32),
        mesh=mesh,
        compiler_params=pltpu.CompilerParams(needs_layout_passes=False),
        scratch_types=[
            pltpu.VMEM((NP,), jnp.float32),
            pltpu.VMEM((EW2,), jnp.int32),
            pltpu.VMEM((EW2,), jnp.int32),
            pltpu.VMEM((EW2,), jnp.float32),
            pltpu.VMEM((NP,), jnp.float32),
            pltpu.VMEM((NS, SW), jnp.float32),
            pltpu.VMEM((SW,), jnp.float32),
            pltpu.VMEM((PADN,), jnp.float32),
            pltpu.VMEM_SHARED((NS, NP), jnp.float32),
            pltpu.VMEM_SHARED((NP,), jnp.float32),
            pltpu.SemaphoreType.DMA,
        ],
    )
    def k(alpha_h, sidx_h, didx_h, attn_h,
          alpha_v, sidx_v, didx_v, aexp_v, dacc_v, redm_v, red_v, zpad_v,
          sp_all, sp_den, isem):
        cid = lax.axis_index("c")
        sid = lax.axis_index("s")
        base = sid * EW2
        lds = [pltpu.async_copy(alpha_h, alpha_v, isem),
               pltpu.async_copy(sidx_h.at[pl.ds(base, EW2)], sidx_v, isem),
               pltpu.async_copy(didx_h.at[pl.ds(base, EW2)], didx_v, isem)]

        def zbody(i, _):
            dacc_v[pl.ds(i * 16, 16)] = jnp.zeros((16,), jnp.float32)
            return 0
        lax.fori_loop(0, NP // 16, zbody, 0, unroll=8)
        for g in range(PADN // 16):
            zpad_v[pl.ds(g * 16, 16)] = jnp.zeros((16,), jnp.float32)
        for d in lds:
            d.wait()

        def ebody(g, _):
            s16 = sidx_v[pl.ds(g * 16, 16)]
            a16 = plsc.load_gather(alpha_v, [s16])
            a16 = jnp.where(a16 >= 0.0, a16, a16 * 0.2)
            x16 = jnp.exp(a16)
            aexp_v[pl.ds(g * 16, 16)] = x16
            d16 = didx_v[pl.ds(g * 16, 16)]
            plsc.addupdate_scatter(dacc_v, [d16], x16)
            return 0
        lax.fori_loop(0, EW2 // 16, ebody, 0, unroll=4)

        pltpu.sync_copy(dacc_v, sp_all.at[sid])
        plsc.subcore_barrier()

        rds = [pltpu.async_copy(
            sp_all.at[t, pl.ds(sid * SW, SW)], redm_v.at[t], isem)
            for t in range(NS)]
        for d in rds:
            d.wait()

        def abody(i, _):
            s = redm_v[0, pl.ds(i * 16, 16)]
            for t in range(1, NS):
                s = s + redm_v[t, pl.ds(i * 16, 16)]
            red_v[pl.ds(i * 16, 16)] = s
            return 0
        lax.fori_loop(0, SW // 16, abody, 0, unroll=4)
        pltpu.sync_copy(red_v, sp_den.at[pl.ds(sid * SW, SW)])
        plsc.subcore_barrier()
        pltpu.sync_copy(sp_den, dacc_v)

        off = cid * EH

        def fbody(g, _):
            x16 = aexp_v[pl.ds(off + g * 16, 16)]
            d16 = didx_v[pl.ds(off + g * 16, 16)]
            dn16 = plsc.load_gather(dacc_v, [d16])
            aexp_v[pl.ds(off + g * 16, 16)] = x16 / (dn16 + 1e-16)
            return 0
        lax.fori_loop(0, EH // 16, fbody, 0, unroll=4)
        blk = (2 * sid + cid) * EWP
        pltpu.sync_copy(
            aexp_v.at[pl.ds(off, EH)], attn_h.at[pl.ds(blk, EH)])
        pltpu.sync_copy(zpad_v, attn_h.at[pl.ds(blk + EH, PADN)])

    return k(alpha, sidx, didx)


def _sc_scatter(xv, attn, sidx, didx):
    EWP = sidx.shape[0] // NW
    NCHK = EWP // CH
    RPB = 2 * NBUF
    CPB = RPB * CH
    mesh = plsc.VectorSubcoreMesh(core_axis_name="c", subcore_axis_name="s")

    @functools.partial(
        pl.kernel,
        out_type=jax.ShapeDtypeStruct((NC * NP, HID), jnp.float32),
        mesh=mesh,
        compiler_params=pltpu.CompilerParams(needs_layout_passes=False),
        scratch_types=(
            [pltpu.VMEM((CH, HID), jnp.float32) for _ in range(NBUF)]
            + [pltpu.VMEM((CPB,), jnp.int32),
               pltpu.VMEM((CPB,), jnp.int32),
               pltpu.VMEM((CPB,), jnp.float32)]
            + [pltpu.VMEM((CH,), jnp.int32) for _ in range(2 * NBUF)]
            + [pltpu.VMEM_SHARED((NP, HID), jnp.float32)]
            + [pltpu.SemaphoreType.DMA for _ in range(2 * NBUF + 1)]),
    )
    def k(xv_h, attn_h, sidx_h, didx_h, outp_h, *rest):
        rows = rest[:NBUF]
        ibuf = rest[NBUF]
        dbuf = rest[NBUF + 1]
        abuf = rest[NBUF + 2]
        didxw = rest[NBUF + 3:2 * NBUF + 3]
        cidxw = rest[2 * NBUF + 3:3 * NBUF + 3]
        sp_out = rest[3 * NBUF + 3]
        gsem = rest[3 * NBUF + 4:4 * NBUF + 4]
        ssem = rest[4 * NBUF + 4:5 * NBUF + 4]
        isem = rest[5 * NBUF + 4]
        cid = lax.axis_index("c")
        sid = lax.axis_index("s")
        base = (cid * NS + sid) * EWP

        def zrow(r, _):
            for f in range(HID // 16):
                rows[0][r, pl.ds(f * 16, 16)] = jnp.zeros((16,), jnp.float32)
            return 0
        lax.fori_loop(0, CH, zrow, 0, unroll=4)
        for kk in range(SW // CH):
            pltpu.sync_copy(rows[0], sp_out.at[pl.ds(sid * SW + kk * CH, CH)])
        plsc.subcore_barrier()

        def set_didxw(r, b):
            for g in range(CH // 16):
                didxw[b][pl.ds(g * 16, 16)] = (
                    dbuf[pl.ds(r * CH + g * 16, 16)])

        def set_cidxw(r, b):
            for g in range(CH // 16):
                cidxw[b][pl.ds(g * 16, 16)] = (
                    ibuf[pl.ds(r * CH + g * 16, 16)])

        def scale(r, b):
            def sbody(j, _):
                av = plsc.load_gather(
                    abuf, [jnp.zeros((16,), jnp.int32) + (r * CH + j)])
                for f in range(HID // 16):
                    rows[b][j, pl.ds(f * 16, 16)] = (
                        rows[b][j, pl.ds(f * 16, 16)] * av)
                return 0
            lax.fori_loop(0, CH, sbody, 0, unroll=4)

        def body(p, _):
            eb = base + p * CPB
            ids = [pltpu.async_copy(sidx_h.at[pl.ds(eb, CPB)], ibuf, isem),
                   pltpu.async_copy(didx_h.at[pl.ds(eb, CPB)], dbuf, isem),
                   pltpu.async_copy(attn_h.at[pl.ds(eb, CPB)], abuf, isem)]
            for d in ids:
                d.wait()
            sds = None
            for w in range(RPB // NBUF):
                gds = []
                for b in range(NBUF):
                    if sds is not None:
                        sds[b].wait()
                    set_cidxw(w * NBUF + b, b)
                    gds.append(pltpu.async_copy(
                        xv_h.at[cidxw[b]], rows[b], gsem[b]))
                sds = []
                for b in range(NBUF):
                    gds[b].wait()
                    scale(w * NBUF + b, b)
                    set_didxw(w * NBUF + b, b)
                    sds.append(pltpu.async_copy(
                        rows[b], sp_out.at[didxw[b]], ssem[b], add=True))
            for d in sds:
                d.wait()
            return 0
        lax.fori_loop(0, NCHK // RPB, body, 0)

        plsc.subcore_barrier()
        pltpu.sync_copy(
            sp_out.at[pl.ds(sid * SW, SW)],
            outp_h.at[pl.ds(cid * NP + sid * SW, SW)])

    return k(xv, attn, sidx, didx)


def kernel(x, edge_index, params):
    src = edge_index[0]
    he = edge_index[1] - jnp.min(edge_index[1])
    E = src.shape[0]
    EW = E // NW
    EWP = ((EW // CH + 7) // 8 * 8) * CH

    def _pad(a, padvals):
        a2 = a.reshape(NW, EW)
        pb = jnp.broadcast_to(padvals, (NW, EWP - EW)).astype(a.dtype)
        return jnp.concatenate([a2, pb], axis=1).reshape(NW * EWP)

    gspread = jnp.arange(EWP - EW, dtype=jnp.int32) % N_NODES
    sspread = NP - (EWP - EW) + jnp.arange(EWP - EW, dtype=jnp.int32)
    src_p = _pad(src, gspread)
    he_p = _pad(he, gspread)
    src_d = _pad(src, sspread)
    he_d = _pad(he, sspread)
    hp = jnp.zeros((NP, x.shape[1]), x.dtype).at[:N_NODES].set(x)
    xv, alpha = _tc_pre(hp, params[0])
    for li, p in enumerate(params):
        s_ids, d_ids = (src, he) if li % 2 == 0 else (he, src)
        s2, d2 = (src_p, he_d) if li % 2 == 0 else (he_p, src_d)
        attn = _sc_softmax(alpha, s_ids, d_ids, EWP)
        outp = _sc_scatter(xv, attn, s2, d2)
        if li < len(params) - 1:
            xv, alpha = _tc_post_pre(outp[:NP], outp[NP:], p, params[li + 1])
        else:
            hp = _tc_post(outp[:NP], outp[NP:], p)
    return hp[:N_NODES]
